# Initial kernel scaffold; baseline (speedup 1.0000x reference)
import jax
import jax.numpy as jnp
from jax.experimental import pallas as pl


def kernel(x, table, pe):
    raise NotImplementedError("write your pallas kernel here")


# speedup vs baseline: 2.2978x; 2.2978x over previous
import functools

import jax
import jax.numpy as jnp
from jax import lax
from jax.experimental import pallas as pl
from jax.experimental.pallas import tpu as pltpu
from jax.experimental.pallas import tpu_sc as plsc

NUM_EMB = 100000
D = 64
S = 200
B = 4096

NC = 2
NS = 16
NW = NC * NS
TOTAL = B * S
PER_W = TOTAL // NW
CHUNK = 128
NCHUNK = PER_W // CHUNK
VEC = 16


def _body(x_hbm, pe2_hbm, table_hbm, out_hbm, idx_v, pe2_v, rows_v, gsem):
    wid = lax.axis_index("s") * NC + lax.axis_index("c")
    base = wid * PER_W

    pltpu.sync_copy(x_hbm.at[pl.ds(base, PER_W)], idx_v)
    pltpu.sync_copy(pe2_hbm, pe2_v)

    def chunk_body(c, carry):
        start = base + c * CHUNK
        s0 = lax.rem(c * CHUNK, S)
        cp = pltpu.async_copy(
            table_hbm.at[idx_v.at[pl.ds(c * CHUNK, CHUNK)]], rows_v, gsem
        )
        cp.wait()

        def add_body(i, carry2):
            for j in range(D // VEC):
                rows_v[i, pl.ds(j * VEC, VEC)] = (
                    rows_v[i, pl.ds(j * VEC, VEC)]
                    + pe2_v[s0 + i, pl.ds(j * VEC, VEC)]
                )
            return carry2

        lax.fori_loop(0, CHUNK, add_body, 0, unroll=4)

        pltpu.sync_copy(rows_v, out_hbm.at[pl.ds(start, CHUNK)])
        return carry

    lax.fori_loop(0, NCHUNK, chunk_body, 0)


@jax.jit
def kernel(x, table, pe):
    xf = x.reshape(TOTAL)
    pe2 = jnp.concatenate([pe, pe], axis=0)

    mesh = plsc.VectorSubcoreMesh(core_axis_name="c", subcore_axis_name="s")
    out_flat = pl.kernel(
        _body,
        out_type=jax.ShapeDtypeStruct((TOTAL, D), jnp.float32),
        mesh=mesh,
        scratch_types=[
            pltpu.VMEM((PER_W,), jnp.int32),
            pltpu.VMEM((2 * S, D), jnp.float32),
            pltpu.VMEM((CHUNK, D), jnp.float32),
            pltpu.SemaphoreType.DMA,
        ],
        compiler_params=pltpu.CompilerParams(use_tc_tiling_on_sc=False),
    )(xf, pe2, table)
    return out_flat.reshape(B, S, D)


# speedup vs baseline: 2.7674x; 1.2044x over previous
import jax
import jax.numpy as jnp
from jax import lax
from jax.experimental import pallas as pl
from jax.experimental.pallas import tpu as pltpu
from jax.experimental.pallas import tpu_sc as plsc

NUM_EMB = 100000
D = 64
S = 200
B = 4096

NC = 2
NS = 16
NW = NC * NS
TOTAL = B * S
PER_W = TOTAL // NW
CHUNK = 128
NCHUNK = PER_W // CHUNK
NBUF = 4
NGRP = NCHUNK // NBUF
VEC = 16


def _body(x_hbm, pe2_hbm, table_hbm, out_hbm,
          idx_v, pe2_v, rows_v, outs_v, gsem, osem):
    wid = lax.axis_index("s") * NC + lax.axis_index("c")
    base = wid * PER_W

    pltpu.sync_copy(x_hbm.at[pl.ds(base, PER_W)], idx_v)
    pltpu.sync_copy(pe2_hbm, pe2_v)

    def fire_gather(c, b):
        pltpu.async_copy(
            table_hbm.at[idx_v.at[pl.ds(c * CHUNK, CHUNK)]],
            rows_v.at[b],
            gsem.at[b],
        )

    for b in range(NBUF):
        fire_gather(b, b)

    def group_body(g, carry):
        for b in range(NBUF):
            c = g * NBUF + b
            start = base + c * CHUNK
            s0 = lax.rem(c * CHUNK, S)

            pltpu.make_async_copy(
                table_hbm.at[idx_v.at[pl.ds(c * CHUNK, CHUNK)]],
                rows_v.at[b],
                gsem.at[b],
            ).wait()

            @pl.when(g > 0)
            def _wait_prev_out():
                pltpu.make_async_copy(
                    outs_v.at[b],
                    out_hbm.at[pl.ds(start, CHUNK)],
                    osem.at[b],
                ).wait()

            def add_body(i, carry2):
                for j in range(D // VEC):
                    outs_v[b, i, pl.ds(j * VEC, VEC)] = (
                        rows_v[b, i, pl.ds(j * VEC, VEC)]
                        + pe2_v[s0 + i, pl.ds(j * VEC, VEC)]
                    )
                return carry2

            lax.fori_loop(0, CHUNK, add_body, 0, unroll=4)

            pltpu.async_copy(
                outs_v.at[b],
                out_hbm.at[pl.ds(start, CHUNK)],
                osem.at[b],
            )

            @pl.when(g < NGRP - 1)
            def _prefetch():
                fire_gather(c + NBUF, b)
        return carry

    lax.fori_loop(0, NGRP, group_body, 0)

    for b in range(NBUF):
        c = (NGRP - 1) * NBUF + b
        pltpu.make_async_copy(
            outs_v.at[b],
            out_hbm.at[pl.ds(base + c * CHUNK, CHUNK)],
            osem.at[b],
        ).wait()


@jax.jit
def kernel(x, table, pe):
    xf = x.reshape(TOTAL)
    pe2 = jnp.concatenate([pe, pe], axis=0)

    mesh = plsc.VectorSubcoreMesh(core_axis_name="c", subcore_axis_name="s")
    out_flat = pl.kernel(
        _body,
        out_type=jax.ShapeDtypeStruct((TOTAL, D), jnp.float32),
        mesh=mesh,
        scratch_types=[
            pltpu.VMEM((PER_W,), jnp.int32),
            pltpu.VMEM((2 * S, D), jnp.float32),
            pltpu.VMEM((NBUF, CHUNK, D), jnp.float32),
            pltpu.VMEM((NBUF, CHUNK, D), jnp.float32),
            pltpu.SemaphoreType.DMA((NBUF,)),
            pltpu.SemaphoreType.DMA((NBUF,)),
        ],
        compiler_params=pltpu.CompilerParams(use_tc_tiling_on_sc=False),
    )(xf, pe2, table)
    return out_flat.reshape(B, S, D)
